# Initial kernel scaffold; baseline (speedup 1.0000x reference)
import jax
import jax.numpy as jnp
from jax.experimental import pallas as pl


def kernel(x, adj, W, att_src, att_dst, bias):
    raise NotImplementedError("write your pallas kernel here")


# speedup vs baseline: 2775.7042x; 2775.7042x over previous
import jax
import jax.numpy as jnp
from jax.experimental import pallas as pl

_TILE = 256


def _dot(a, b, dims):
    return jax.lax.dot_general(
        a, b, (dims, ((), ())),
        precision=jax.lax.Precision.HIGHEST,
        preferred_element_type=jnp.float32,
    )


def _gat_kernel(x_ref, xt_ref, adj_ref, wf_ref, asrc_ref, adstt_ref, bias_ref,
                out_ref):
    x = x_ref[...]
    wf = wf_ref[...]
    h_all = _dot(x, wf, (((1,), (0,))))
    a_src = _dot(h_all, asrc_ref[...], (((1,), (0,))))
    h_tile = _dot(xt_ref[...], wf, (((1,), (0,))))
    adt = _dot(adstt_ref[...], h_tile, (((1,), (1,))))
    n_heads = adt.shape[0]
    out_ch = h_all.shape[1] // n_heads
    mask = adj_ref[...] > 0.5
    neg_inf = jnp.float32(-jnp.inf)
    for h in range(n_heads):
        e = a_src[:, h:h + 1] + adt[h:h + 1, :]
        e = jnp.where(e >= 0, e, 0.2 * e)
        em = jnp.where(mask, e, neg_inf)
        m = jnp.max(em, axis=0, keepdims=True)
        m = jnp.where(jnp.isfinite(m), m, 0.0)
        p = jnp.exp(em - m)
        denom = jnp.sum(p, axis=0, keepdims=True)
        coef = p / (denom + 1e-16)
        hh = h_all[:, h * out_ch:(h + 1) * out_ch]
        ot = _dot(coef, hh, (((0,), (0,))))
        out_ref[:, h * out_ch:(h + 1) * out_ch] = (
            ot + bias_ref[:, h * out_ch:(h + 1) * out_ch])


def kernel(x, adj, W, att_src, att_dst, bias):
    n, d_in = x.shape
    heads, out_ch = att_src.shape
    wf = W.reshape(d_in, heads * out_ch)
    eye = jnp.eye(heads, dtype=jnp.float32)
    asrc = (eye[:, None, :] * att_src[:, :, None]).reshape(heads * out_ch, heads)
    adstt = (eye[:, :, None] * att_dst[None, :, :]).reshape(heads, heads * out_ch)
    bias2 = bias.reshape(1, heads * out_ch)
    grid = (n // _TILE,)
    return pl.pallas_call(
        _gat_kernel,
        grid=grid,
        in_specs=[
            pl.BlockSpec((n, d_in), lambda j: (0, 0)),
            pl.BlockSpec((_TILE, d_in), lambda j: (j, 0)),
            pl.BlockSpec((n, _TILE), lambda j: (0, j)),
            pl.BlockSpec((d_in, heads * out_ch), lambda j: (0, 0)),
            pl.BlockSpec((heads * out_ch, heads), lambda j: (0, 0)),
            pl.BlockSpec((heads, heads * out_ch), lambda j: (0, 0)),
            pl.BlockSpec((1, heads * out_ch), lambda j: (0, 0)),
        ],
        out_specs=pl.BlockSpec((_TILE, heads * out_ch), lambda j: (j, 0)),
        out_shape=jax.ShapeDtypeStruct((n, heads * out_ch), jnp.float32),
    )(x, x, adj, wf, asrc, adstt, bias2)
